# Initial kernel scaffold; baseline (speedup 1.0000x reference)
import jax
import jax.numpy as jnp
from jax.experimental import pallas as pl


def kernel(x, shared_fc, shared_proj, experts_fc, experts_proj, gate_w, expert_bias):
    raise NotImplementedError("write your pallas kernel here")


# speedup vs baseline: 1.6392x; 1.6392x over previous
import functools

import jax
import jax.numpy as jnp
from jax.experimental import pallas as pl
from jax.experimental.pallas import tpu as pltpu

_TOP_K = 4


def _gmm_kernel(off_r, end_r, tid_r, eid_r, x_ref, fcg_ref, fcx_ref,
                proj_ref, out_ref, *, tm):
    g = pl.program_id(0)
    h = pl.program_id(1)
    xb = x_ref[...]
    gg = jnp.dot(xb, fcg_ref[0], preferred_element_type=jnp.float32)
    hh = jnp.dot(xb, fcx_ref[0], preferred_element_type=jnp.float32)
    act = (gg * jax.nn.sigmoid(gg)) * hh
    row = tid_r[g] * tm + jax.lax.broadcasted_iota(jnp.int32, (tm, 1), 0)
    mask = (row >= off_r[g]) & (row < end_r[g])
    act = jnp.where(mask, act, 0.0)
    contrib = jnp.dot(act, proj_ref[0], preferred_element_type=jnp.float32)
    prev_tid = tid_r[jnp.maximum(g - 1, 0)]
    first = (h == 0) & ((g == 0) | (tid_r[g] != prev_tid))

    @pl.when(first)
    def _():
        out_ref[...] = contrib

    @pl.when(jnp.logical_not(first))
    def _():
        out_ref[...] += contrib


def _dense_ffn_kernel(x_ref, fcg_ref, fcx_ref, proj_ref, out_ref):
    h = pl.program_id(1)
    xb = x_ref[...]
    gg = jnp.dot(xb, fcg_ref[...], preferred_element_type=jnp.float32)
    hh = jnp.dot(xb, fcx_ref[...], preferred_element_type=jnp.float32)
    act = (gg * jax.nn.sigmoid(gg)) * hh
    contrib = jnp.dot(act, proj_ref[...], preferred_element_type=jnp.float32)

    @pl.when(h == 0)
    def _():
        out_ref[...] = contrib

    @pl.when(h != 0)
    def _():
        out_ref[...] += contrib


def kernel(x, shared_fc, shared_proj, experts_fc, experts_proj, gate_w,
           expert_bias):
    Bq, Tq, C = x.shape
    E = experts_fc.shape[0]
    HID = experts_proj.shape[1]
    K = _TOP_K
    N = Bq * Tq
    S = N * K
    i32 = jnp.int32

    TM = min(512, S)
    HB = min(512, HID)
    assert S % TM == 0 and HID % HB == 0
    NT = S // TM
    NH = HID // HB
    G = NT + E - 1

    flat_x = x.reshape(N, C)

    logits = flat_x @ gate_w + expert_bias
    gw = jax.nn.sigmoid(logits)
    top_w, top_i = jax.lax.top_k(gw, K)
    top_w = top_w / jnp.sum(top_w, axis=-1, keepdims=True)
    e_flat = top_i.reshape(-1).astype(i32)

    order = jnp.argsort(e_flat, stable=True).astype(i32)
    tok_sorted = order // K
    x_sorted = jnp.take(flat_x, tok_sorted, axis=0)

    sizes = jnp.bincount(e_flat, length=E).astype(i32)
    offsets = jnp.concatenate(
        [jnp.zeros((1,), i32), jnp.cumsum(sizes).astype(i32)])
    first_tile = offsets[:E] // TM
    last_tile = (offsets[1:] - 1) // TM
    n_t = jnp.where(sizes > 0, last_tile - first_tile + 1, 0).astype(i32)
    cum_nt = jnp.cumsum(n_t)
    items_before = cum_nt - n_t
    total = cum_nt[-1]

    i = jnp.arange(G, dtype=i32)
    e_of = jnp.searchsorted(cum_nt, i, side='right').astype(i32)
    valid = i < total
    e_idx = jnp.minimum(e_of, E - 1)
    tile_ids = jnp.where(valid, first_tile[e_idx] + (i - items_before[e_idx]),
                         NT - 1).astype(i32)
    expert_ids = jnp.where(valid, e_idx, 0).astype(i32)
    off_arr = jnp.where(valid, offsets[e_idx], S).astype(i32)
    end_arr = jnp.where(valid, offsets[e_idx + 1], S).astype(i32)

    gmm = pl.pallas_call(
        functools.partial(_gmm_kernel, tm=TM),
        grid_spec=pltpu.PrefetchScalarGridSpec(
            num_scalar_prefetch=4,
            grid=(G, NH),
            in_specs=[
                pl.BlockSpec((TM, C),
                             lambda g, h, off, end, tid, eid: (tid[g], 0)),
                pl.BlockSpec((1, C, HB),
                             lambda g, h, off, end, tid, eid: (eid[g], 0, h)),
                pl.BlockSpec((1, C, HB),
                             lambda g, h, off, end, tid, eid:
                             (eid[g], 0, h + NH)),
                pl.BlockSpec((1, HB, C),
                             lambda g, h, off, end, tid, eid: (eid[g], h, 0)),
            ],
            out_specs=pl.BlockSpec((TM, C),
                                   lambda g, h, off, end, tid, eid:
                                   (tid[g], 0)),
        ),
        out_shape=jax.ShapeDtypeStruct((S, C), jnp.float32),
    )
    out_sorted = gmm(off_arr, end_arr, tile_ids, expert_ids, x_sorted,
                     experts_fc, experts_fc, experts_proj)

    TMS = min(512, N)
    NTS = N // TMS
    dense = pl.pallas_call(
        _dense_ffn_kernel,
        grid=(NTS, NH),
        in_specs=[
            pl.BlockSpec((TMS, C), lambda t, h: (t, 0)),
            pl.BlockSpec((C, HB), lambda t, h: (0, h)),
            pl.BlockSpec((C, HB), lambda t, h: (0, h + NH)),
            pl.BlockSpec((HB, C), lambda t, h: (h, 0)),
        ],
        out_specs=pl.BlockSpec((TMS, C), lambda t, h: (t, 0)),
        out_shape=jax.ShapeDtypeStruct((N, C), jnp.float32),
    )
    shared_out = dense(flat_x, shared_fc, shared_fc, shared_proj)

    inv = jnp.zeros((S,), i32).at[order].set(jnp.arange(S, dtype=i32))
    routed = jnp.sum(
        out_sorted[inv.reshape(N, K)] * top_w[..., None], axis=1)

    return (shared_out + routed).reshape(Bq, Tq, C)


# speedup vs baseline: 1.6504x; 1.0068x over previous
import functools

import jax
import jax.numpy as jnp
from jax.experimental import pallas as pl
from jax.experimental.pallas import tpu as pltpu

_TOP_K = 4


def _gmm_kernel(off_r, end_r, tid_r, eid_r, x_ref, fcg_ref, fcx_ref,
                proj_ref, out_ref, *, tm):
    g = pl.program_id(0)
    h = pl.program_id(1)
    xb = x_ref[...].astype(jnp.bfloat16)
    gg = jnp.dot(xb, fcg_ref[0].astype(jnp.bfloat16),
                 preferred_element_type=jnp.float32)
    hh = jnp.dot(xb, fcx_ref[0].astype(jnp.bfloat16),
                 preferred_element_type=jnp.float32)
    act = (gg * jax.nn.sigmoid(gg)) * hh
    row = tid_r[g] * tm + jax.lax.broadcasted_iota(jnp.int32, (tm, 1), 0)
    mask = (row >= off_r[g]) & (row < end_r[g])
    act = jnp.where(mask, act, 0.0).astype(jnp.bfloat16)
    contrib = jnp.dot(act, proj_ref[0].astype(jnp.bfloat16),
                      preferred_element_type=jnp.float32)
    prev_tid = tid_r[jnp.maximum(g - 1, 0)]
    first = (h == 0) & ((g == 0) | (tid_r[g] != prev_tid))

    @pl.when(first)
    def _():
        out_ref[...] = contrib

    @pl.when(jnp.logical_not(first))
    def _():
        out_ref[...] += contrib


def _dense_ffn_kernel(x_ref, fcg_ref, fcx_ref, proj_ref, out_ref):
    h = pl.program_id(1)
    xb = x_ref[...].astype(jnp.bfloat16)
    gg = jnp.dot(xb, fcg_ref[...].astype(jnp.bfloat16),
                 preferred_element_type=jnp.float32)
    hh = jnp.dot(xb, fcx_ref[...].astype(jnp.bfloat16),
                 preferred_element_type=jnp.float32)
    act = ((gg * jax.nn.sigmoid(gg)) * hh).astype(jnp.bfloat16)
    contrib = jnp.dot(act, proj_ref[...].astype(jnp.bfloat16),
                      preferred_element_type=jnp.float32)

    @pl.when(h == 0)
    def _():
        out_ref[...] = contrib

    @pl.when(h != 0)
    def _():
        out_ref[...] += contrib


def kernel(x, shared_fc, shared_proj, experts_fc, experts_proj, gate_w,
           expert_bias):
    Bq, Tq, C = x.shape
    E = experts_fc.shape[0]
    HID = experts_proj.shape[1]
    K = _TOP_K
    N = Bq * Tq
    S = N * K
    i32 = jnp.int32

    TM = min(512, S)
    HB = min(512, HID)
    assert S % TM == 0 and HID % HB == 0
    NT = S // TM
    NH = HID // HB
    G = NT + E - 1

    flat_x = x.reshape(N, C)

    logits = flat_x @ gate_w + expert_bias
    gw = jax.nn.sigmoid(logits)
    top_w, top_i = jax.lax.top_k(gw, K)
    top_w = top_w / jnp.sum(top_w, axis=-1, keepdims=True)
    e_flat = top_i.reshape(-1).astype(i32)

    order = jnp.argsort(e_flat, stable=True).astype(i32)
    tok_sorted = order // K
    x_sorted = jnp.take(flat_x, tok_sorted, axis=0)

    sizes = jnp.bincount(e_flat, length=E).astype(i32)
    offsets = jnp.concatenate(
        [jnp.zeros((1,), i32), jnp.cumsum(sizes).astype(i32)])
    first_tile = offsets[:E] // TM
    last_tile = (offsets[1:] - 1) // TM
    n_t = jnp.where(sizes > 0, last_tile - first_tile + 1, 0).astype(i32)
    cum_nt = jnp.cumsum(n_t)
    items_before = cum_nt - n_t
    total = cum_nt[-1]

    i = jnp.arange(G, dtype=i32)
    e_of = jnp.searchsorted(cum_nt, i, side='right').astype(i32)
    valid = i < total
    e_idx = jnp.minimum(e_of, E - 1)
    tile_ids = jnp.where(valid, first_tile[e_idx] + (i - items_before[e_idx]),
                         NT - 1).astype(i32)
    expert_ids = jnp.where(valid, e_idx, 0).astype(i32)
    off_arr = jnp.where(valid, offsets[e_idx], S).astype(i32)
    end_arr = jnp.where(valid, offsets[e_idx + 1], S).astype(i32)

    gmm = pl.pallas_call(
        functools.partial(_gmm_kernel, tm=TM),
        grid_spec=pltpu.PrefetchScalarGridSpec(
            num_scalar_prefetch=4,
            grid=(G, NH),
            in_specs=[
                pl.BlockSpec((TM, C),
                             lambda g, h, off, end, tid, eid: (tid[g], 0)),
                pl.BlockSpec((1, C, HB),
                             lambda g, h, off, end, tid, eid: (eid[g], 0, h)),
                pl.BlockSpec((1, C, HB),
                             lambda g, h, off, end, tid, eid:
                             (eid[g], 0, h + NH)),
                pl.BlockSpec((1, HB, C),
                             lambda g, h, off, end, tid, eid: (eid[g], h, 0)),
            ],
            out_specs=pl.BlockSpec((TM, C),
                                   lambda g, h, off, end, tid, eid:
                                   (tid[g], 0)),
        ),
        out_shape=jax.ShapeDtypeStruct((S, C), jnp.float32),
    )
    out_sorted = gmm(off_arr, end_arr, tile_ids, expert_ids, x_sorted,
                     experts_fc, experts_fc, experts_proj)

    TMS = min(512, N)
    NTS = N // TMS
    dense = pl.pallas_call(
        _dense_ffn_kernel,
        grid=(NTS, NH),
        in_specs=[
            pl.BlockSpec((TMS, C), lambda t, h: (t, 0)),
            pl.BlockSpec((C, HB), lambda t, h: (0, h)),
            pl.BlockSpec((C, HB), lambda t, h: (0, h + NH)),
            pl.BlockSpec((HB, C), lambda t, h: (h, 0)),
        ],
        out_specs=pl.BlockSpec((TMS, C), lambda t, h: (t, 0)),
        out_shape=jax.ShapeDtypeStruct((N, C), jnp.float32),
    )
    shared_out = dense(flat_x, shared_fc, shared_fc, shared_proj)

    inv = jnp.zeros((S,), i32).at[order].set(jnp.arange(S, dtype=i32))
    routed = jnp.sum(
        out_sorted[inv.reshape(N, K)] * top_w[..., None], axis=1)

    return (shared_out + routed).reshape(Bq, Tq, C)
